# baseline (speedup 1.0000x reference)
import functools

import jax
import jax.numpy as jnp
import numpy as np
from jax import lax
from jax.experimental import pallas as pl
from jax.experimental.pallas import tpu as pltpu
from jax.experimental.pallas import tpu_sc as plsc

N = 10000
E = 320000
D = 128
HD = D // 2
G = 64
C = 10
EPS = 1e-5

NP = 10240
ROWS_PT = NP // 16
CH = 80
NCH = 130
E2 = 32 * NCH * CH
NBUF = 5
NOUT = NCH // NBUF

_PB = np.arange(32).reshape(16, 2).T.reshape(32)
_P = np.concatenate([32 * j + _PB for j in range(D // 32)])

_mesh = plsc.VectorSubcoreMesh(core_axis_name="c", subcore_axis_name="s")
_params = pltpu.CompilerParams(use_tc_tiling_on_sc=False)


@functools.partial(
    pl.kernel,
    mesh=_mesh,
    out_type=jax.ShapeDtypeStruct((2, NP), jnp.float32),
    scratch_types=[
        pltpu.VMEM((NCH, CH), jnp.int32),
        pltpu.VMEM((NCH, CH), jnp.float32),
        pltpu.VMEM_SHARED((NP,), jnp.float32),
    ] + [pltpu.SemaphoreType.DMA] * NBUF,
    compiler_params=_params,
)
def _deg_sc(ei_hbm, ew3_hbm, zrow_hbm, out_hbm, idx_v, val_v, acc_sh,
            *ssems):
    cid = lax.axis_index("c")
    sid = lax.axis_index("s")
    r0 = sid * ROWS_PT
    pltpu.sync_copy(zrow_hbm, acc_sh.at[pl.ds(r0, ROWS_PT)])
    tid = cid * 16 + sid
    pltpu.sync_copy(ei_hbm.at[1, tid], idx_v)
    pltpu.sync_copy(ew3_hbm.at[tid], val_v)
    plsc.subcore_barrier()

    def issue(io, _):
        for b in range(NBUF):
            i = io * NBUF + b
            pltpu.async_copy(val_v.at[i], acc_sh.at[idx_v.at[i]], ssems[b],
                             add=True)
        return _

    lax.fori_loop(0, NOUT, issue, None)

    def drain(io, _):
        for b in range(NBUF):
            i = io * NBUF + b
            pltpu.make_async_copy(val_v.at[i], acc_sh.at[idx_v.at[i]],
                                  ssems[b]).wait()
        return _

    lax.fori_loop(0, NOUT, drain, None)
    plsc.subcore_barrier()
    pltpu.sync_copy(acc_sh.at[pl.ds(r0, ROWS_PT)],
                    out_hbm.at[cid, pl.ds(r0, ROWS_PT)])


@functools.partial(
    pl.kernel,
    mesh=_mesh,
    out_type=jax.ShapeDtypeStruct((2, NP, HD), jnp.float32),
    scratch_types=[
        pltpu.VMEM((NCH, CH), jnp.int32),
        pltpu.VMEM((NCH, CH), jnp.int32),
        pltpu.VMEM((NCH, CH), jnp.float32),
    ] + [pltpu.VMEM((CH, HD), jnp.float32)] * NBUF
      + [pltpu.VMEM_SHARED((NP, HD), jnp.float32)]
      + [pltpu.SemaphoreType.DMA] * (2 * NBUF),
    compiler_params=_params,
)
def _msg_sc(table_hbm, ei_hbm, ew3_hbm, zrows_hbm, out_hbm,
            idxr_v, idxc_v, ew_v, r0b, r1b, r2b, r3b, r4b, acc_sh, *sems):
    rows = [r0b, r1b, r2b, r3b, r4b]
    gsems = sems[:NBUF]
    ssems = sems[NBUF:]
    cid = lax.axis_index("c")
    sid = lax.axis_index("s")
    r0 = sid * ROWS_PT
    pltpu.sync_copy(zrows_hbm, acc_sh.at[pl.ds(r0, ROWS_PT)])
    tabc = table_hbm.at[cid]

    def start_gather(c, b):
        pltpu.async_copy(tabc.at[idxr_v.at[c]], rows[b], gsems[b])

    def wait_gather(b):
        pltpu.make_async_copy(tabc.at[idxr_v.at[0]], rows[b],
                              gsems[b]).wait()

    def start_scatter(j, b):
        pltpu.async_copy(rows[b], acc_sh.at[idxc_v.at[j]], ssems[b], add=True)

    def wait_scatter(b):
        pltpu.make_async_copy(rows[b], acc_sh.at[idxc_v.at[0]],
                              ssems[b]).wait()

    def scale(j, b):
        rb = rows[b]

        @plsc.parallel_loop(0, CH // 16)
        def grp(g):
            wv = ew_v[j, pl.ds(g * 16, 16)]
            for k2 in range(16):
                w = jnp.full((16,), wv[k2], jnp.float32)
                k = g * 16 + k2
                for d8 in range(HD // 16):
                    sl = pl.ds(d8 * 16, 16)
                    rb[k, sl] = rb[k, sl] * w

    plsc.subcore_barrier()

    def process_half(half, _h):
        erow = sid + 16 * half
        pltpu.sync_copy(ei_hbm.at[0, erow], idxr_v)
        pltpu.sync_copy(ei_hbm.at[1, erow], idxc_v)
        pltpu.sync_copy(ew3_hbm.at[erow], ew_v)

        for b in range(NBUF - 1):
            start_gather(b, b)

        def body(io, _):
            for b in range(NBUF):
                j = io * NBUF + b
                wait_gather(b)
                scale(j, b)
                start_scatter(j, b)
                b4 = (b + 4) % NBUF

                @pl.when(j >= 1)
                def _w():
                    wait_scatter(b4)

                @pl.when(j <= NCH - NBUF)
                def _g():
                    start_gather(j + 4, b4)
            return _

        lax.fori_loop(0, NOUT, body, None)
        wait_scatter((NCH - 1) % NBUF)
        return _h

    lax.fori_loop(0, 2, process_half, None)

    plsc.subcore_barrier()
    pltpu.sync_copy(acc_sh.at[pl.ds(r0, ROWS_PT)],
                    out_hbm.at[cid, pl.ds(r0, ROWS_PT)])


def _tc1_body(x_ref, w1_ref, degp_ref, t_ref, dinv_ref):
    deg = degp_ref[0, :N] + degp_ref[1, :N]
    dinv = jnp.where(deg > 0, lax.rsqrt(deg), 0.0)
    h1 = jnp.dot(x_ref[...], w1_ref[...], preferred_element_type=jnp.float32)
    h1s = h1 * dinv[:, None]
    t_ref[0] = h1s[:, :HD]
    t_ref[1] = h1s[:, HD:]
    dinv_ref[...] = dinv[:, None]


def _tc2_body(sp_ref, dinv_ref, b_ref, g_ref, be_ref, w2_ref, t2_ref):
    s = jnp.concatenate([sp_ref[0, :N, :], sp_ref[1, :N, :]], axis=1)
    dinv = dinv_ref[...]
    z = dinv * s + b_ref[...]
    mu = jnp.mean(z, axis=0, keepdims=True)
    var = jnp.mean((z - mu) * (z - mu), axis=0, keepdims=True)
    zn = (z - mu) * lax.rsqrt(var + EPS) * g_ref[...] + be_ref[...]
    h = jnp.maximum(zn, 0.0)
    h2 = jnp.dot(h, w2_ref[...], preferred_element_type=jnp.float32)
    h2s = h2 * dinv
    t2_ref[0] = h2s[:, :HD]
    t2_ref[1] = h2s[:, HD:]


def _tc3_body(sp_ref, dinv_ref, b_ref, g_ref, be_ref, batch_ref,
              wl_ref, bl_ref, out_ref):
    s = jnp.concatenate([sp_ref[0, :N, :], sp_ref[1, :N, :]], axis=1)
    dinv = dinv_ref[...]
    z = dinv * s + b_ref[...]
    mu = jnp.mean(z, axis=0, keepdims=True)
    var = jnp.mean((z - mu) * (z - mu), axis=0, keepdims=True)
    zn = (z - mu) * lax.rsqrt(var + EPS) * g_ref[...] + be_ref[...]
    h = jnp.maximum(zn, 0.0)
    gi = lax.broadcasted_iota(jnp.int32, (N, G), 1)
    oh = (batch_ref[...] == gi).astype(jnp.float32)
    cnt = jnp.sum(oh, axis=0)
    ssum = lax.dot_general(oh, h, (((0,), (0,)), ((), ())),
                           preferred_element_type=jnp.float32)
    pooled = ssum / jnp.maximum(cnt, 1.0)[:, None]
    out_ref[...] = jnp.dot(pooled, wl_ref[...],
                           preferred_element_type=jnp.float32) + bl_ref[...]


def kernel(x, edge_index, edge_attr, batch, W1, b1, gamma1, beta1,
           W2, b2, gamma2, beta2, Wl, bl):
    pad = E2 - E - N
    loop = jnp.arange(N, dtype=edge_index.dtype)
    ei = jnp.concatenate(
        [edge_index, jnp.stack([loop, loop]),
         jnp.zeros((2, pad), edge_index.dtype)], axis=1).reshape(2, 32, NCH, CH)
    ew3 = jnp.concatenate(
        [edge_attr, jnp.ones((N,), edge_attr.dtype),
         jnp.zeros((pad,), edge_attr.dtype)]).reshape(32, NCH, CH)
    zrow = jnp.zeros((ROWS_PT,), jnp.float32)
    zrows = jnp.zeros((ROWS_PT, HD), jnp.float32)

    degp = _deg_sc(ei, ew3, zrow)

    t1, dinv = pl.pallas_call(
        _tc1_body,
        out_shape=[jax.ShapeDtypeStruct((2, N, HD), jnp.float32),
                   jax.ShapeDtypeStruct((N, 1), jnp.float32)],
    )(x, W1, degp)

    s1p = _msg_sc(t1, ei, ew3, zrows)

    t2 = pl.pallas_call(
        _tc2_body,
        out_shape=jax.ShapeDtypeStruct((2, N, HD), jnp.float32),
    )(s1p, dinv, b1[None, :], gamma1[None, :], beta1[None, :], W2)

    s2p = _msg_sc(t2, ei, ew3, zrows)

    out = pl.pallas_call(
        _tc3_body,
        out_shape=jax.ShapeDtypeStruct((G, C), jnp.float32),
    )(s2p, dinv, b2[None, :], gamma2[None, :], beta2[None, :],
      batch[:, None], Wl, bl[None, :])
    return out


# speedup vs baseline: 1.4108x; 1.4108x over previous
import functools

import jax
import jax.numpy as jnp
from jax import lax
from jax.experimental import pallas as pl
from jax.experimental.pallas import tpu as pltpu
from jax.experimental.pallas import tpu_sc as plsc

N = 10000
E = 320000
D = 128
HD = D // 2
G = 64
C = 10
EPS = 1e-5

NP = 10240
ROWS_PT = NP // 16
CH = 80
NCH = 125
NBUF = 5
NOUT = NCH // NBUF

_mesh = plsc.VectorSubcoreMesh(core_axis_name="c", subcore_axis_name="s")
_params = pltpu.CompilerParams(use_tc_tiling_on_sc=False)


@functools.partial(
    pl.kernel,
    mesh=_mesh,
    out_type=jax.ShapeDtypeStruct((2, NP), jnp.float32),
    scratch_types=[
        pltpu.VMEM((NCH, CH), jnp.int32),
        pltpu.VMEM((NCH, CH), jnp.float32),
        pltpu.VMEM_SHARED((NP,), jnp.float32),
    ] + [pltpu.SemaphoreType.DMA] * NBUF,
    compiler_params=_params,
)
def _deg_sc(ei_hbm, ew3_hbm, zrow_hbm, out_hbm, idx_v, val_v, acc_sh,
            *ssems):
    cid = lax.axis_index("c")
    sid = lax.axis_index("s")
    r0 = sid * ROWS_PT
    pltpu.sync_copy(zrow_hbm, acc_sh.at[pl.ds(r0, ROWS_PT)])
    tid = cid * 16 + sid
    pltpu.sync_copy(ei_hbm.at[1, tid], idx_v)
    pltpu.sync_copy(ew3_hbm.at[tid], val_v)
    plsc.subcore_barrier()

    def issue(io, _):
        for b in range(NBUF):
            i = io * NBUF + b
            pltpu.async_copy(val_v.at[i], acc_sh.at[idx_v.at[i]], ssems[b],
                             add=True)
        return _

    lax.fori_loop(0, NOUT, issue, None)

    def drain(io, _):
        for b in range(NBUF):
            i = io * NBUF + b
            pltpu.make_async_copy(val_v.at[i], acc_sh.at[idx_v.at[i]],
                                  ssems[b]).wait()
        return _

    lax.fori_loop(0, NOUT, drain, None)
    plsc.subcore_barrier()
    pltpu.sync_copy(acc_sh.at[pl.ds(r0, ROWS_PT)],
                    out_hbm.at[cid, pl.ds(r0, ROWS_PT)])


@functools.partial(
    pl.kernel,
    mesh=_mesh,
    out_type=jax.ShapeDtypeStruct((2, NP, HD), jnp.float32),
    scratch_types=[
        pltpu.VMEM((NCH, CH), jnp.int32),
        pltpu.VMEM((NCH, CH), jnp.int32),
        pltpu.VMEM((NCH, CH), jnp.float32),
    ] + [pltpu.VMEM((CH, HD), jnp.float32)] * NBUF
      + [pltpu.VMEM_SHARED((NP, HD), jnp.float32)]
      + [pltpu.SemaphoreType.DMA] * (2 * NBUF),
    compiler_params=_params,
)
def _msg_sc(table_hbm, ei_hbm, ew3_hbm, zrows_hbm, out_hbm,
            idxr_v, idxc_v, ew_v, r0b, r1b, r2b, r3b, r4b, acc_sh, *sems):
    rows = [r0b, r1b, r2b, r3b, r4b]
    gsems = sems[:NBUF]
    ssems = sems[NBUF:]
    cid = lax.axis_index("c")
    sid = lax.axis_index("s")
    r0 = sid * ROWS_PT
    pltpu.sync_copy(zrows_hbm, acc_sh.at[pl.ds(r0, ROWS_PT)])
    tabc = table_hbm.at[cid]

    def start_gather(c, b):
        pltpu.async_copy(tabc.at[idxr_v.at[c]], rows[b], gsems[b])

    def wait_gather(b):
        pltpu.make_async_copy(tabc.at[idxr_v.at[0]], rows[b], gsems[b]).wait()

    def start_scatter(j, b):
        pltpu.async_copy(rows[b], acc_sh.at[idxc_v.at[j]], ssems[b], add=True)

    def wait_scatter(b):
        pltpu.make_async_copy(rows[b], acc_sh.at[idxc_v.at[0]],
                              ssems[b]).wait()

    def scale(j, b):
        rb = rows[b]

        @plsc.parallel_loop(0, CH // 16)
        def grp(g):
            wv = ew_v[j, pl.ds(g * 16, 16)]
            for k2 in range(16):
                w = jnp.full((16,), wv[k2], jnp.float32)
                k = g * 16 + k2
                for d8 in range(HD // 16):
                    sl = pl.ds(d8 * 16, 16)
                    rb[k, sl] = rb[k, sl] * w

    plsc.subcore_barrier()

    def process_half(half, _h):
        erow = sid + 16 * half
        pltpu.sync_copy(ei_hbm.at[0, erow], idxr_v)
        pltpu.sync_copy(ei_hbm.at[1, erow], idxc_v)
        pltpu.sync_copy(ew3_hbm.at[erow], ew_v)

        for b in range(NBUF - 1):
            start_gather(b, b)

        def body(io, _):
            for b in range(NBUF):
                j = io * NBUF + b
                wait_gather(b)
                scale(j, b)
                start_scatter(j, b)
                b4 = (b + 4) % NBUF

                @pl.when(j >= 1)
                def _w():
                    wait_scatter(b4)

                @pl.when(j <= NCH - NBUF)
                def _g():
                    start_gather(j + 4, b4)
            return _

        lax.fori_loop(0, NOUT, body, None)
        wait_scatter((NCH - 1) % NBUF)
        return _h

    lax.fori_loop(0, 2, process_half, None)

    plsc.subcore_barrier()
    pltpu.sync_copy(acc_sh.at[pl.ds(r0, ROWS_PT)],
                    out_hbm.at[cid, pl.ds(r0, ROWS_PT)])


def _dinv(degp_ref):
    deg = degp_ref[0, :N] + degp_ref[1, :N] + 1.0
    return jnp.where(deg > 0, lax.rsqrt(deg), 0.0)


def _tc1_body(x_ref, w1_ref, degp_ref, t_ref):
    dinv = _dinv(degp_ref)
    h1 = jnp.dot(x_ref[...], w1_ref[...], preferred_element_type=jnp.float32)
    h1s = h1 * dinv[:, None]
    t_ref[0] = h1s[:, :HD]
    t_ref[1] = h1s[:, HD:]


def _tc2_body(sp_ref, t_ref, degp_ref, b_ref, g_ref, be_ref, w2_ref, t2_ref):
    dinv = _dinv(degp_ref)[:, None]
    s = jnp.concatenate([sp_ref[0, :N, :], sp_ref[1, :N, :]], axis=1)
    hs = jnp.concatenate([t_ref[0], t_ref[1]], axis=1)
    z = dinv * (s + hs) + b_ref[...]
    mu = jnp.mean(z, axis=0, keepdims=True)
    var = jnp.mean((z - mu) * (z - mu), axis=0, keepdims=True)
    zn = (z - mu) * lax.rsqrt(var + EPS) * g_ref[...] + be_ref[...]
    h = jnp.maximum(zn, 0.0)
    h2 = jnp.dot(h, w2_ref[...], preferred_element_type=jnp.float32)
    h2s = h2 * dinv
    t2_ref[0] = h2s[:, :HD]
    t2_ref[1] = h2s[:, HD:]


def _tc3_body(sp_ref, t_ref, degp_ref, b_ref, g_ref, be_ref, batch_ref,
              wl_ref, bl_ref, out_ref):
    dinv = _dinv(degp_ref)[:, None]
    s = jnp.concatenate([sp_ref[0, :N, :], sp_ref[1, :N, :]], axis=1)
    hs = jnp.concatenate([t_ref[0], t_ref[1]], axis=1)
    z = dinv * (s + hs) + b_ref[...]
    mu = jnp.mean(z, axis=0, keepdims=True)
    var = jnp.mean((z - mu) * (z - mu), axis=0, keepdims=True)
    zn = (z - mu) * lax.rsqrt(var + EPS) * g_ref[...] + be_ref[...]
    h = jnp.maximum(zn, 0.0)
    gi = lax.broadcasted_iota(jnp.int32, (G, N), 0)
    oht = (batch_ref[...] == gi).astype(jnp.float32)
    cnt = jnp.sum(oht, axis=1)
    ssum = jnp.dot(oht, h, preferred_element_type=jnp.float32)
    pooled = ssum / jnp.maximum(cnt, 1.0)[:, None]
    out_ref[...] = jnp.dot(pooled, wl_ref[...],
                           preferred_element_type=jnp.float32) + bl_ref[...]


def kernel(x, edge_index, edge_attr, batch, W1, b1, gamma1, beta1,
           W2, b2, gamma2, beta2, Wl, bl):
    ei = edge_index.reshape(2, 32, NCH, CH)
    ew3 = edge_attr.reshape(32, NCH, CH)
    zrow = jnp.zeros((ROWS_PT,), jnp.float32)
    zrows = jnp.zeros((ROWS_PT, HD), jnp.float32)

    degp = _deg_sc(ei, ew3, zrow)

    t1 = pl.pallas_call(
        _tc1_body,
        out_shape=jax.ShapeDtypeStruct((2, N, HD), jnp.float32),
    )(x, W1, degp)

    s1p = _msg_sc(t1, ei, ew3, zrows)

    t2 = pl.pallas_call(
        _tc2_body,
        out_shape=jax.ShapeDtypeStruct((2, N, HD), jnp.float32),
    )(s1p, t1, degp, b1[None, :], gamma1[None, :], beta1[None, :], W2)

    s2p = _msg_sc(t2, ei, ew3, zrows)

    out = pl.pallas_call(
        _tc3_body,
        out_shape=jax.ShapeDtypeStruct((G, C), jnp.float32),
    )(s2p, t2, degp, b2[None, :], gamma2[None, :], beta2[None, :],
      batch[None, :], Wl, bl[None, :])
    return out
